# Initial kernel scaffold; baseline (speedup 1.0000x reference)
import jax
import jax.numpy as jnp
from jax.experimental import pallas as pl


def kernel(x, edge_index, edge_attr, WQ, WK, WV, We):
    raise NotImplementedError("write your pallas kernel here")


# speedup vs baseline: 9.0042x; 9.0042x over previous
import jax
import jax.numpy as jnp
import numpy as np
from jax import lax
from jax.experimental import pallas as pl
from jax.experimental.pallas import tpu as pltpu
from jax.experimental.pallas import tpu_sc as plsc

N = 10000
E = 320000
D = 128
H = 8
DH = 16

C = 64
NW = 32
NCHUNK = E // C
JMAX = -(-NCHUNK // NW)
NTILE = 16
ROWS_PER_TILE = 624
TAIL_ROW = NTILE * ROWS_PER_TILE
TAIL_ROWS = N - TAIL_ROW

_ZB = np.zeros((16, 128), np.float32)
for _h in range(H):
    _ZB[_h, _h * DH:(_h + 1) * DH] = 1.0
_ZB.setflags(write=False)


def _mm_nodes_body(x_ref, wk_ref, wq_ref, wv_ref, k_ref, q_ref, v_ref):
    x = x_ref[...]
    k_ref[...] = jnp.dot(x, wk_ref[...], preferred_element_type=jnp.float32)
    q_ref[...] = jnp.dot(x, wq_ref[...], preferred_element_type=jnp.float32)
    v_ref[...] = jnp.dot(x, wv_ref[...], preferred_element_type=jnp.float32)


def _mm_edges_body(ea_ref, we_ref, pe_ref):
    pe_ref[...] = jnp.dot(ea_ref[...], we_ref[...],
                          preferred_element_type=jnp.float32)


def _final_body(w_ref, z_ref, zb_ref, out_ref):
    w = w_ref[0] + w_ref[1]
    z = z_ref[0] + z_ref[1]
    zbig = jnp.dot(z, zb_ref[...], preferred_element_type=jnp.float32)
    out_ref[...] = w / (zbig + 1e-6)


def _edge_body(k_hbm, q_hbm, v_hbm, pe_hbm, src_hbm, dst_hbm, zw_hbm, zz_hbm,
               outw_hbm, outz_hbm,
               k_v, q_v, pe_v, src_v, dst_v, msg_v, s_v,
               accw, accz, sem):
    cid = lax.axis_index("c")
    sid = lax.axis_index("s")
    wid = sid * 2 + cid

    pltpu.sync_copy(zw_hbm, msg_v)
    pltpu.sync_copy(zz_hbm, s_v)
    r0 = sid * ROWS_PER_TILE
    for j in range(15):
        pltpu.sync_copy(msg_v, accw.at[pl.ds(r0 + j * C, C)])
        pltpu.sync_copy(s_v, accz.at[pl.ds(r0 + j * C, C)])
    pltpu.sync_copy(msg_v.at[pl.ds(0, 24)], accw.at[pl.ds(r0 + 960, 24)])
    pltpu.sync_copy(s_v.at[pl.ds(0, 24)], accz.at[pl.ds(r0 + 960, 24)])

    @pl.when(sid == 0)
    def _():
        pltpu.sync_copy(msg_v.at[pl.ds(0, TAIL_ROWS)],
                        accw.at[pl.ds(TAIL_ROW, TAIL_ROWS)])
        pltpu.sync_copy(s_v.at[pl.ds(0, TAIL_ROWS)],
                        accz.at[pl.ds(TAIL_ROW, TAIL_ROWS)])
    plsc.subcore_barrier()

    rows16 = lax.iota(jnp.int32, 16)

    def chunk_body(j, carry):
        c = j * NW + wid

        @pl.when(c < NCHUNK)
        def _():
            base = c * C
            pltpu.sync_copy(src_hbm.at[pl.ds(base, C)], src_v)
            pltpu.sync_copy(dst_hbm.at[pl.ds(base, C)], dst_v)
            pltpu.async_copy(k_hbm.at[src_v], k_v, sem).wait()
            pltpu.async_copy(q_hbm.at[dst_v], q_v, sem).wait()
            pltpu.async_copy(v_hbm.at[src_v], msg_v, sem).wait()
            pltpu.sync_copy(pe_hbm.at[pl.ds(base, C)], pe_v)

            def group_body(g, gcarry):
                r = rows16 + g * 16
                for h in range(H):
                    t = jnp.zeros((16,), jnp.float32)
                    for dh in range(DH):
                        ci = jnp.full((16,), h * DH + dh, jnp.int32)
                        rk = plsc.load_gather(k_v, [r, ci])
                        rq = plsc.load_gather(q_v, [r, ci])
                        rp = plsc.load_gather(pe_v, [r, ci])
                        t = t + rk * rq * rp
                    s = jnp.exp(jnp.clip(t, -5.0, 5.0))
                    plsc.store_scatter(
                        s_v, [r, jnp.full((16,), h, jnp.int32)], s)
                    for dh in range(DH):
                        ci = jnp.full((16,), h * DH + dh, jnp.int32)
                        rv = plsc.load_gather(msg_v, [r, ci])
                        plsc.store_scatter(msg_v, [r, ci], rv * s)
                return gcarry

            lax.fori_loop(0, C // 16, group_body, 0)

            pltpu.sync_copy(msg_v, accw.at[dst_v], add=True)
            pltpu.sync_copy(s_v, accz.at[dst_v], add=True)

        return carry

    lax.fori_loop(0, JMAX, chunk_body, 0)
    plsc.subcore_barrier()

    pltpu.sync_copy(accw.at[pl.ds(r0, ROWS_PER_TILE)],
                    outw_hbm.at[cid, pl.ds(r0, ROWS_PER_TILE)])
    pltpu.sync_copy(accz.at[pl.ds(r0, ROWS_PER_TILE)],
                    outz_hbm.at[cid, pl.ds(r0, ROWS_PER_TILE)])

    @pl.when(sid == 0)
    def _():
        pltpu.sync_copy(accw.at[pl.ds(TAIL_ROW, TAIL_ROWS)],
                        outw_hbm.at[cid, pl.ds(TAIL_ROW, TAIL_ROWS)])
        pltpu.sync_copy(accz.at[pl.ds(TAIL_ROW, TAIL_ROWS)],
                        outz_hbm.at[cid, pl.ds(TAIL_ROW, TAIL_ROWS)])


_edge_call = pl.kernel(
    _edge_body,
    out_type=(jax.ShapeDtypeStruct((2, N, 128), jnp.float32),
              jax.ShapeDtypeStruct((2, N, 16), jnp.float32)),
    mesh=plsc.VectorSubcoreMesh(core_axis_name="c", subcore_axis_name="s"),
    compiler_params=pltpu.CompilerParams(use_tc_tiling_on_sc=False,
                                         needs_layout_passes=False),
    scratch_types=[
        pltpu.VMEM((C, 128), jnp.float32),
        pltpu.VMEM((C, 128), jnp.float32),
        pltpu.VMEM((C, 128), jnp.float32),
        pltpu.VMEM((C,), jnp.int32),
        pltpu.VMEM((C,), jnp.int32),
        pltpu.VMEM((C, 128), jnp.float32),
        pltpu.VMEM((C, 16), jnp.float32),
        pltpu.VMEM_SHARED((N, 128), jnp.float32),
        pltpu.VMEM_SHARED((N, 16), jnp.float32),
        pltpu.SemaphoreType.DMA,
    ],
)


def kernel(x, edge_index, edge_attr, WQ, WK, WV, We):
    src = edge_index[0]
    dst = edge_index[1]
    wq = WQ * (1.0 / np.sqrt(np.float32(DH)))

    bn = 2000
    k, q, v = pl.pallas_call(
        _mm_nodes_body,
        grid=(N // bn,),
        in_specs=[
            pl.BlockSpec((bn, D), lambda i: (i, 0)),
            pl.BlockSpec((D, 128), lambda i: (0, 0)),
            pl.BlockSpec((D, 128), lambda i: (0, 0)),
            pl.BlockSpec((D, 128), lambda i: (0, 0)),
        ],
        out_specs=[
            pl.BlockSpec((bn, 128), lambda i: (i, 0)),
            pl.BlockSpec((bn, 128), lambda i: (i, 0)),
            pl.BlockSpec((bn, 128), lambda i: (i, 0)),
        ],
        out_shape=[
            jax.ShapeDtypeStruct((N, 128), jnp.float32),
            jax.ShapeDtypeStruct((N, 128), jnp.float32),
            jax.ShapeDtypeStruct((N, 128), jnp.float32),
        ],
    )(x, WK, wq, WV)

    be = 3200
    pe = pl.pallas_call(
        _mm_edges_body,
        grid=(E // be,),
        in_specs=[
            pl.BlockSpec((be, D), lambda i: (i, 0)),
            pl.BlockSpec((D, 128), lambda i: (0, 0)),
        ],
        out_specs=pl.BlockSpec((be, 128), lambda i: (i, 0)),
        out_shape=jax.ShapeDtypeStruct((E, 128), jnp.float32),
    )(edge_attr, We)

    zw = jnp.zeros((C, 128), jnp.float32)
    zz = jnp.zeros((C, 16), jnp.float32)
    outw, outz = _edge_call(k, q, v, pe, src, dst, zw, zz)

    bf = 2000
    out = pl.pallas_call(
        _final_body,
        grid=(N // bf,),
        in_specs=[
            pl.BlockSpec((2, bf, 128), lambda i: (0, i, 0)),
            pl.BlockSpec((2, bf, 16), lambda i: (0, i, 0)),
            pl.BlockSpec((16, 128), lambda i: (0, 0)),
        ],
        out_specs=pl.BlockSpec((bf, 128), lambda i: (i, 0)),
        out_shape=jax.ShapeDtypeStruct((N, 128), jnp.float32),
    )(outw, outz, jnp.asarray(_ZB))

    return out


# speedup vs baseline: 9.5467x; 1.0603x over previous
import jax
import jax.numpy as jnp
import numpy as np
from jax import lax
from jax.experimental import pallas as pl
from jax.experimental.pallas import tpu as pltpu
from jax.experimental.pallas import tpu_sc as plsc

N = 10000
E = 320000
D = 128
H = 8
DH = 16
HH = H // 2
CW = HH * DH

C = 128
NT = 16
NCHUNK = E // C
JMAX = -(-NCHUNK // (2 * NT))
ROWS_PER_TILE = 624
TAIL_ROW = NT * ROWS_PER_TILE
TAIL_ROWS = N - TAIL_ROW

_ZB = np.zeros((32, 128), np.float32)
for _c in range(2):
    for _h in range(HH):
        _g = _c * HH + _h
        _ZB[_c * 16 + _h, _g * DH:(_g + 1) * DH] = 1.0
_ZB.setflags(write=False)


def _mm_nodes_body(x_ref, wk_ref, wq_ref, wv_ref, k_ref, q_ref, v_ref):
    x = x_ref[...]
    for w_ref, o_ref in ((wk_ref, k_ref), (wq_ref, q_ref), (wv_ref, v_ref)):
        r = jnp.dot(x, w_ref[...], preferred_element_type=jnp.float32)
        o_ref[0] = r[:, :CW]
        o_ref[1] = r[:, CW:]


def _mm_edges_body(ea_ref, we_ref, pe_ref):
    r = jnp.dot(ea_ref[...], we_ref[...], preferred_element_type=jnp.float32)
    pe_ref[0] = r[:, :CW]
    pe_ref[1] = r[:, CW:]


def _final_body(w_ref, z_ref, zb_ref, out_ref):
    w = jnp.concatenate([w_ref[0], w_ref[1]], axis=1)
    z = jnp.concatenate([z_ref[0], z_ref[1]], axis=1)
    zbig = jnp.dot(z, zb_ref[...], preferred_element_type=jnp.float32)
    out_ref[...] = w / (zbig + 1e-6)


def _edge_body(k_hbm, q_hbm, v_hbm, pe_hbm, src_hbm, dst_hbm, zw_hbm, zz_hbm,
               outw_hbm, outz_hbm,
               k_v0, k_v1, q_v0, q_v1, pe_v0, pe_v1, msg_v0, msg_v1,
               s_v0, s_v1, src_v0, src_v1, dst_v0, dst_v1,
               accw, accz, sem0, sem1):
    cid = lax.axis_index("c")
    sid = lax.axis_index("s")
    kt = k_hbm.at[cid]
    qt = q_hbm.at[cid]
    vt = v_hbm.at[cid]
    pet = pe_hbm.at[cid]
    bufs = ((k_v0, q_v0, pe_v0, msg_v0, s_v0, src_v0, dst_v0, sem0),
            (k_v1, q_v1, pe_v1, msg_v1, s_v1, src_v1, dst_v1, sem1))

    pltpu.sync_copy(zw_hbm, msg_v0)
    pltpu.sync_copy(zz_hbm, s_v0)
    r0 = sid * ROWS_PER_TILE
    for j in range(4):
        pltpu.sync_copy(msg_v0, accw.at[pl.ds(r0 + j * C, C)])
        pltpu.sync_copy(s_v0, accz.at[pl.ds(r0 + j * C, C)])
    pltpu.sync_copy(msg_v0.at[pl.ds(0, 112)], accw.at[pl.ds(r0 + 512, 112)])
    pltpu.sync_copy(s_v0.at[pl.ds(0, 112)], accz.at[pl.ds(r0 + 512, 112)])

    @pl.when(sid == 0)
    def _():
        pltpu.sync_copy(msg_v0.at[pl.ds(0, TAIL_ROWS)],
                        accw.at[pl.ds(TAIL_ROW, TAIL_ROWS)])
        pltpu.sync_copy(s_v0.at[pl.ds(0, TAIL_ROWS)],
                        accz.at[pl.ds(TAIL_ROW, TAIL_ROWS)])
    plsc.subcore_barrier()

    rows16 = lax.iota(jnp.int32, 16)

    def issue(b, c):
        k_v, q_v, pe_v, msg_v, s_v, src_v, dst_v, sem = bufs[b]
        base = c * C
        pltpu.sync_copy(src_hbm.at[pl.ds(base, C)], src_v)
        pltpu.sync_copy(dst_hbm.at[pl.ds(base, C)], dst_v)
        pltpu.async_copy(kt.at[src_v], k_v, sem)
        pltpu.async_copy(qt.at[dst_v], q_v, sem)
        pltpu.async_copy(vt.at[src_v], msg_v, sem)
        pltpu.async_copy(pet.at[pl.ds(base, C)], pe_v, sem)

    def wait(b, c):
        k_v, q_v, pe_v, msg_v, s_v, src_v, dst_v, sem = bufs[b]
        base = c * C
        pltpu.make_async_copy(kt.at[src_v], k_v, sem).wait()
        pltpu.make_async_copy(qt.at[dst_v], q_v, sem).wait()
        pltpu.make_async_copy(vt.at[src_v], msg_v, sem).wait()
        pltpu.make_async_copy(pet.at[pl.ds(base, C)], pe_v, sem).wait()

    def compute(b):
        k_v, q_v, pe_v, msg_v, s_v, src_v, dst_v, sem = bufs[b]

        def group_body(g, gcarry):
            r = rows16 + g * 16
            for h in range(HH):
                t = jnp.zeros((16,), jnp.float32)
                for dh in range(DH):
                    ci = jnp.full((16,), h * DH + dh, jnp.int32)
                    rk = plsc.load_gather(k_v, [r, ci])
                    rq = plsc.load_gather(q_v, [r, ci])
                    rp = plsc.load_gather(pe_v, [r, ci])
                    t = t + rk * rq * rp
                s = jnp.exp(jnp.clip(t, -5.0, 5.0))
                plsc.store_scatter(
                    s_v, [r, jnp.full((16,), h, jnp.int32)], s)
                for dh in range(DH):
                    ci = jnp.full((16,), h * DH + dh, jnp.int32)
                    rv = plsc.load_gather(msg_v, [r, ci])
                    plsc.store_scatter(msg_v, [r, ci], rv * s)
            return gcarry

        lax.fori_loop(0, C // 16, group_body, 0)
        pltpu.sync_copy(msg_v, accw.at[dst_v], add=True)
        pltpu.sync_copy(s_v, accz.at[dst_v], add=True)

    for b in range(2):
        issue(b, b * NT + sid)

    def chunk_body(j, carry):
        for b in range(2):
            c = (2 * j + b) * NT + sid
            cn = c + 2 * NT

            @pl.when(c < NCHUNK)
            def _():
                wait(b, c)
                compute(b)

                @pl.when(cn < NCHUNK)
                def _():
                    issue(b, cn)

        return carry

    lax.fori_loop(0, JMAX, chunk_body, 0)
    plsc.subcore_barrier()

    pltpu.sync_copy(accw.at[pl.ds(r0, ROWS_PER_TILE)],
                    outw_hbm.at[cid, pl.ds(r0, ROWS_PER_TILE)])
    pltpu.sync_copy(accz.at[pl.ds(r0, ROWS_PER_TILE)],
                    outz_hbm.at[cid, pl.ds(r0, ROWS_PER_TILE)])

    @pl.when(sid == 0)
    def _():
        pltpu.sync_copy(accw.at[pl.ds(TAIL_ROW, TAIL_ROWS)],
                        outw_hbm.at[cid, pl.ds(TAIL_ROW, TAIL_ROWS)])
        pltpu.sync_copy(accz.at[pl.ds(TAIL_ROW, TAIL_ROWS)],
                        outz_hbm.at[cid, pl.ds(TAIL_ROW, TAIL_ROWS)])


_edge_call = pl.kernel(
    _edge_body,
    out_type=(jax.ShapeDtypeStruct((2, N, CW), jnp.float32),
              jax.ShapeDtypeStruct((2, N, 16), jnp.float32)),
    mesh=plsc.VectorSubcoreMesh(core_axis_name="c", subcore_axis_name="s"),
    compiler_params=pltpu.CompilerParams(use_tc_tiling_on_sc=False,
                                         needs_layout_passes=False),
    scratch_types=[
        pltpu.VMEM((C, CW), jnp.float32),
        pltpu.VMEM((C, CW), jnp.float32),
        pltpu.VMEM((C, CW), jnp.float32),
        pltpu.VMEM((C, CW), jnp.float32),
        pltpu.VMEM((C, CW), jnp.float32),
        pltpu.VMEM((C, CW), jnp.float32),
        pltpu.VMEM((C, CW), jnp.float32),
        pltpu.VMEM((C, CW), jnp.float32),
        pltpu.VMEM((C, 16), jnp.float32),
        pltpu.VMEM((C, 16), jnp.float32),
        pltpu.VMEM((C,), jnp.int32),
        pltpu.VMEM((C,), jnp.int32),
        pltpu.VMEM((C,), jnp.int32),
        pltpu.VMEM((C,), jnp.int32),
        pltpu.VMEM_SHARED((N, CW), jnp.float32),
        pltpu.VMEM_SHARED((N, 16), jnp.float32),
        pltpu.SemaphoreType.DMA,
        pltpu.SemaphoreType.DMA,
    ],
)


def kernel(x, edge_index, edge_attr, WQ, WK, WV, We):
    src = edge_index[0]
    dst = edge_index[1]
    wq = WQ * (1.0 / np.sqrt(np.float32(DH)))

    bn = 2000
    k, q, v = pl.pallas_call(
        _mm_nodes_body,
        grid=(N // bn,),
        in_specs=[
            pl.BlockSpec((bn, D), lambda i: (i, 0)),
            pl.BlockSpec((D, 128), lambda i: (0, 0)),
            pl.BlockSpec((D, 128), lambda i: (0, 0)),
            pl.BlockSpec((D, 128), lambda i: (0, 0)),
        ],
        out_specs=[
            pl.BlockSpec((2, bn, CW), lambda i: (0, i, 0)),
            pl.BlockSpec((2, bn, CW), lambda i: (0, i, 0)),
            pl.BlockSpec((2, bn, CW), lambda i: (0, i, 0)),
        ],
        out_shape=[
            jax.ShapeDtypeStruct((2, N, CW), jnp.float32),
            jax.ShapeDtypeStruct((2, N, CW), jnp.float32),
            jax.ShapeDtypeStruct((2, N, CW), jnp.float32),
        ],
    )(x, WK, wq, WV)

    be = 3200
    pe = pl.pallas_call(
        _mm_edges_body,
        grid=(E // be,),
        in_specs=[
            pl.BlockSpec((be, D), lambda i: (i, 0)),
            pl.BlockSpec((D, 128), lambda i: (0, 0)),
        ],
        out_specs=pl.BlockSpec((2, be, CW), lambda i: (0, i, 0)),
        out_shape=jax.ShapeDtypeStruct((2, E, CW), jnp.float32),
    )(edge_attr, We)

    zw = jnp.zeros((C, CW), jnp.float32)
    zz = jnp.zeros((C, 16), jnp.float32)
    outw, outz = _edge_call(k, q, v, pe, src, dst, zw, zz)

    bf = 2000
    out = pl.pallas_call(
        _final_body,
        grid=(N // bf,),
        in_specs=[
            pl.BlockSpec((2, bf, CW), lambda i: (0, i, 0)),
            pl.BlockSpec((2, bf, 16), lambda i: (0, i, 0)),
            pl.BlockSpec((32, 128), lambda i: (0, 0)),
        ],
        out_specs=pl.BlockSpec((bf, 128), lambda i: (i, 0)),
        out_shape=jax.ShapeDtypeStruct((N, 128), jnp.float32),
    )(outw, outz, jnp.asarray(_ZB))

    return out


# speedup vs baseline: 42.2472x; 4.4253x over previous
import jax
import jax.numpy as jnp
import numpy as np
from jax import lax
from jax.experimental import pallas as pl
from jax.experimental.pallas import tpu as pltpu
from jax.experimental.pallas import tpu_sc as plsc

N = 10000
E = 320000
D = 128
H = 8
DH = 16
HH = H // 2
CW = HH * DH

C = 128
NT = 16
NCHUNK = E // C
JMAX = -(-NCHUNK // (2 * NT))
ROWS_PER_TILE = 624
TAIL_ROW = NT * ROWS_PER_TILE
TAIL_ROWS = N - TAIL_ROW

_ZB = np.zeros((32, 128), np.float32)
for _c in range(2):
    for _h in range(HH):
        _g = _c * HH + _h
        _ZB[_c * 16 + _h, _g * DH:(_g + 1) * DH] = 1.0
_ZB.setflags(write=False)


def _mm_nodes_body(x_ref, wk_ref, wq_ref, wv_ref, k_ref, q_ref, v_ref):
    x = x_ref[...]
    for w_ref, o_ref in ((wk_ref, k_ref), (wq_ref, q_ref), (wv_ref, v_ref)):
        r = jnp.dot(x, w_ref[...], preferred_element_type=jnp.float32)
        o_ref[0] = r[:, :CW]
        o_ref[1] = r[:, CW:]


def _mm_edges_body(ea_ref, we_ref, pe_ref):
    r = jnp.dot(ea_ref[...], we_ref[...], preferred_element_type=jnp.float32)
    pe_ref[0] = r[:, :CW]
    pe_ref[1] = r[:, CW:]


def _final_body(w_ref, z_ref, zb_ref, out_ref):
    w = jnp.concatenate([w_ref[0], w_ref[1]], axis=1)
    z = jnp.concatenate([z_ref[0], z_ref[1]], axis=1)
    zbig = jnp.dot(z, zb_ref[...], preferred_element_type=jnp.float32)
    out_ref[...] = w / (zbig + 1e-6)


def _edge_body(k_hbm, q_hbm, v_hbm, pe_hbm, src_hbm, dst_hbm, zw_hbm, zz_hbm,
               outw_hbm, outz_hbm,
               k_v0, k_v1, q_v0, q_v1, pe_v0, pe_v1, msg_v0, msg_v1,
               s_v0, s_v1, src_v0, src_v1, dst_v0, dst_v1,
               accw, accz, sem0, sem1):
    cid = lax.axis_index("c")
    sid = lax.axis_index("s")
    kt = k_hbm.at[cid]
    qt = q_hbm.at[cid]
    vt = v_hbm.at[cid]
    pet = pe_hbm.at[cid]
    bufs = ((k_v0, q_v0, pe_v0, msg_v0, s_v0, src_v0, dst_v0, sem0),
            (k_v1, q_v1, pe_v1, msg_v1, s_v1, src_v1, dst_v1, sem1))

    pltpu.sync_copy(zw_hbm, msg_v0)
    pltpu.sync_copy(zz_hbm, s_v0)
    r0 = sid * ROWS_PER_TILE
    for j in range(4):
        pltpu.sync_copy(msg_v0, accw.at[pl.ds(r0 + j * C, C)])
        pltpu.sync_copy(s_v0, accz.at[pl.ds(r0 + j * C, C)])
    pltpu.sync_copy(msg_v0.at[pl.ds(0, 112)], accw.at[pl.ds(r0 + 512, 112)])
    pltpu.sync_copy(s_v0.at[pl.ds(0, 112)], accz.at[pl.ds(r0 + 512, 112)])

    @pl.when(sid == 0)
    def _():
        pltpu.sync_copy(msg_v0.at[pl.ds(0, TAIL_ROWS)],
                        accw.at[pl.ds(TAIL_ROW, TAIL_ROWS)])
        pltpu.sync_copy(s_v0.at[pl.ds(0, TAIL_ROWS)],
                        accz.at[pl.ds(TAIL_ROW, TAIL_ROWS)])
    plsc.subcore_barrier()

    rows16 = lax.iota(jnp.int32, 16)

    def issue(b, c):
        k_v, q_v, pe_v, msg_v, s_v, src_v, dst_v, sem = bufs[b]
        base = c * C
        pltpu.sync_copy(src_hbm.at[pl.ds(base, C)], src_v)
        pltpu.sync_copy(dst_hbm.at[pl.ds(base, C)], dst_v)
        pltpu.async_copy(kt.at[src_v], k_v, sem)
        pltpu.async_copy(qt.at[dst_v], q_v, sem)
        pltpu.async_copy(vt.at[src_v], msg_v, sem)
        pltpu.async_copy(pet.at[pl.ds(base, C)], pe_v, sem)

    def wait(b, c):
        k_v, q_v, pe_v, msg_v, s_v, src_v, dst_v, sem = bufs[b]
        base = c * C
        pltpu.make_async_copy(kt.at[src_v], k_v, sem).wait()
        pltpu.make_async_copy(qt.at[dst_v], q_v, sem).wait()
        pltpu.make_async_copy(vt.at[src_v], msg_v, sem).wait()
        pltpu.make_async_copy(pet.at[pl.ds(base, C)], pe_v, sem).wait()

    def compute(b):
        k_v, q_v, pe_v, msg_v, s_v, src_v, dst_v, sem = bufs[b]

        @plsc.parallel_loop(0, C, unroll=4)
        def _(e):
            srow = jnp.zeros((16,), jnp.float32)
            for h in range(HH):
                sl = pl.ds(h * DH, DH)
                t = jnp.sum(k_v[e, sl] * q_v[e, sl] * pe_v[e, sl])
                sv = jnp.exp(jnp.clip(jnp.full((16,), t), -5.0, 5.0))
                msg_v[e, sl] = msg_v[e, sl] * sv
                srow = jnp.where(rows16 == h, sv, srow)
            s_v[e, :] = srow

        pltpu.sync_copy(msg_v, accw.at[dst_v], add=True)
        pltpu.sync_copy(s_v, accz.at[dst_v], add=True)

    for b in range(2):
        issue(b, b * NT + sid)

    def chunk_body(j, carry):
        for b in range(2):
            c = (2 * j + b) * NT + sid
            cn = c + 2 * NT

            @pl.when(c < NCHUNK)
            def _():
                wait(b, c)
                compute(b)

                @pl.when(cn < NCHUNK)
                def _():
                    issue(b, cn)

        return carry

    lax.fori_loop(0, JMAX, chunk_body, 0)
    plsc.subcore_barrier()

    pltpu.sync_copy(accw.at[pl.ds(r0, ROWS_PER_TILE)],
                    outw_hbm.at[cid, pl.ds(r0, ROWS_PER_TILE)])
    pltpu.sync_copy(accz.at[pl.ds(r0, ROWS_PER_TILE)],
                    outz_hbm.at[cid, pl.ds(r0, ROWS_PER_TILE)])

    @pl.when(sid == 0)
    def _():
        pltpu.sync_copy(accw.at[pl.ds(TAIL_ROW, TAIL_ROWS)],
                        outw_hbm.at[cid, pl.ds(TAIL_ROW, TAIL_ROWS)])
        pltpu.sync_copy(accz.at[pl.ds(TAIL_ROW, TAIL_ROWS)],
                        outz_hbm.at[cid, pl.ds(TAIL_ROW, TAIL_ROWS)])


_edge_call = pl.kernel(
    _edge_body,
    out_type=(jax.ShapeDtypeStruct((2, N, CW), jnp.float32),
              jax.ShapeDtypeStruct((2, N, 16), jnp.float32)),
    mesh=plsc.VectorSubcoreMesh(core_axis_name="c", subcore_axis_name="s"),
    compiler_params=pltpu.CompilerParams(use_tc_tiling_on_sc=False,
                                         needs_layout_passes=False),
    scratch_types=[
        pltpu.VMEM((C, CW), jnp.float32),
        pltpu.VMEM((C, CW), jnp.float32),
        pltpu.VMEM((C, CW), jnp.float32),
        pltpu.VMEM((C, CW), jnp.float32),
        pltpu.VMEM((C, CW), jnp.float32),
        pltpu.VMEM((C, CW), jnp.float32),
        pltpu.VMEM((C, CW), jnp.float32),
        pltpu.VMEM((C, CW), jnp.float32),
        pltpu.VMEM((C, 16), jnp.float32),
        pltpu.VMEM((C, 16), jnp.float32),
        pltpu.VMEM((C,), jnp.int32),
        pltpu.VMEM((C,), jnp.int32),
        pltpu.VMEM((C,), jnp.int32),
        pltpu.VMEM((C,), jnp.int32),
        pltpu.VMEM_SHARED((N, CW), jnp.float32),
        pltpu.VMEM_SHARED((N, 16), jnp.float32),
        pltpu.SemaphoreType.DMA,
        pltpu.SemaphoreType.DMA,
    ],
)


def kernel(x, edge_index, edge_attr, WQ, WK, WV, We):
    src = edge_index[0]
    dst = edge_index[1]
    wq = WQ * (1.0 / np.sqrt(np.float32(DH)))

    bn = 2000
    k, q, v = pl.pallas_call(
        _mm_nodes_body,
        grid=(N // bn,),
        in_specs=[
            pl.BlockSpec((bn, D), lambda i: (i, 0)),
            pl.BlockSpec((D, 128), lambda i: (0, 0)),
            pl.BlockSpec((D, 128), lambda i: (0, 0)),
            pl.BlockSpec((D, 128), lambda i: (0, 0)),
        ],
        out_specs=[
            pl.BlockSpec((2, bn, CW), lambda i: (0, i, 0)),
            pl.BlockSpec((2, bn, CW), lambda i: (0, i, 0)),
            pl.BlockSpec((2, bn, CW), lambda i: (0, i, 0)),
        ],
        out_shape=[
            jax.ShapeDtypeStruct((2, N, CW), jnp.float32),
            jax.ShapeDtypeStruct((2, N, CW), jnp.float32),
            jax.ShapeDtypeStruct((2, N, CW), jnp.float32),
        ],
    )(x, WK, wq, WV)

    be = 3200
    pe = pl.pallas_call(
        _mm_edges_body,
        grid=(E // be,),
        in_specs=[
            pl.BlockSpec((be, D), lambda i: (i, 0)),
            pl.BlockSpec((D, 128), lambda i: (0, 0)),
        ],
        out_specs=pl.BlockSpec((2, be, CW), lambda i: (0, i, 0)),
        out_shape=jax.ShapeDtypeStruct((2, E, CW), jnp.float32),
    )(edge_attr, We)

    zw = jnp.zeros((C, CW), jnp.float32)
    zz = jnp.zeros((C, 16), jnp.float32)
    outw, outz = _edge_call(k, q, v, pe, src, dst, zw, zz)

    bf = 2000
    out = pl.pallas_call(
        _final_body,
        grid=(N // bf,),
        in_specs=[
            pl.BlockSpec((2, bf, CW), lambda i: (0, i, 0)),
            pl.BlockSpec((2, bf, 16), lambda i: (0, i, 0)),
            pl.BlockSpec((32, 128), lambda i: (0, 0)),
        ],
        out_specs=pl.BlockSpec((bf, 128), lambda i: (i, 0)),
        out_shape=jax.ShapeDtypeStruct((N, 128), jnp.float32),
    )(outw, outz, jnp.asarray(_ZB))

    return out


# speedup vs baseline: 59.9083x; 1.4180x over previous
import jax
import jax.numpy as jnp
import numpy as np
from jax import lax
from jax.experimental import pallas as pl
from jax.experimental.pallas import tpu as pltpu
from jax.experimental.pallas import tpu_sc as plsc

N = 10000
E = 320000
D = 128
H = 8
DH = 16
HH = H // 2
CW = HH * DH

C = 128
NT = 16
NCHUNK = E // C
JMAX = -(-NCHUNK // (2 * NT))
ROWS_PER_TILE = 624
TAIL_ROW = NT * ROWS_PER_TILE
TAIL_ROWS = N - TAIL_ROW

_ZB = np.zeros((32, 128), np.float32)
for _c in range(2):
    for _h in range(HH):
        _g = _c * HH + _h
        _ZB[_c * 16 + _h, _g * DH:(_g + 1) * DH] = 1.0
_ZB.setflags(write=False)


def _mm_nodes_body(x_ref, wk_ref, wq_ref, wv_ref, k_ref, q_ref, v_ref):
    x = x_ref[...]
    for w_ref, o_ref in ((wk_ref, k_ref), (wq_ref, q_ref), (wv_ref, v_ref)):
        r = jnp.dot(x, w_ref[...], preferred_element_type=jnp.float32)
        o_ref[0] = r[:, :CW]
        o_ref[1] = r[:, CW:]


def _mm_edges_body(ea_ref, we_ref, pe_ref):
    pe_ref[...] = jnp.dot(ea_ref[...], we_ref[...],
                          preferred_element_type=jnp.float32)


def _final_body(w_ref, z_ref, zb_ref, out_ref):
    w = jnp.concatenate([w_ref[0], w_ref[1]], axis=1)
    z = jnp.concatenate([z_ref[0], z_ref[1]], axis=1)
    zbig = jnp.dot(z, zb_ref[...], preferred_element_type=jnp.float32)
    out_ref[...] = w / (zbig + 1e-6)


def _edge_body(k_hbm, q_hbm, v_hbm, pe_hbm, src_hbm, dst_hbm, zw_hbm, zz_hbm,
               outw_hbm, outz_hbm,
               k_v0, k_v1, q_v0, q_v1, pe_v0, pe_v1, msg_v0, msg_v1,
               s_v0, s_v1, src_v0, src_v1, dst_v0, dst_v1,
               accw, accz, sem0, sem1):
    cid = lax.axis_index("c")
    sid = lax.axis_index("s")
    kt = k_hbm.at[cid]
    qt = q_hbm.at[cid]
    vt = v_hbm.at[cid]
    col0 = pl.multiple_of(cid * CW, CW)
    bufs = ((k_v0, q_v0, pe_v0, msg_v0, s_v0, src_v0, dst_v0, sem0),
            (k_v1, q_v1, pe_v1, msg_v1, s_v1, src_v1, dst_v1, sem1))

    pltpu.sync_copy(zw_hbm, msg_v0)
    pltpu.sync_copy(zz_hbm, s_v0)
    r0 = sid * ROWS_PER_TILE
    for j in range(4):
        pltpu.sync_copy(msg_v0, accw.at[pl.ds(r0 + j * C, C)])
        pltpu.sync_copy(s_v0, accz.at[pl.ds(r0 + j * C, C)])
    pltpu.sync_copy(msg_v0.at[pl.ds(0, 112)], accw.at[pl.ds(r0 + 512, 112)])
    pltpu.sync_copy(s_v0.at[pl.ds(0, 112)], accz.at[pl.ds(r0 + 512, 112)])

    @pl.when(sid == 0)
    def _():
        pltpu.sync_copy(msg_v0.at[pl.ds(0, TAIL_ROWS)],
                        accw.at[pl.ds(TAIL_ROW, TAIL_ROWS)])
        pltpu.sync_copy(s_v0.at[pl.ds(0, TAIL_ROWS)],
                        accz.at[pl.ds(TAIL_ROW, TAIL_ROWS)])
    plsc.subcore_barrier()

    rows16 = lax.iota(jnp.int32, 16)

    def issue(b, c):
        k_v, q_v, pe_v, msg_v, s_v, src_v, dst_v, sem = bufs[b]
        base = c * C
        pltpu.sync_copy(src_hbm.at[pl.ds(base, C)], src_v)
        pltpu.sync_copy(dst_hbm.at[pl.ds(base, C)], dst_v)
        pltpu.async_copy(kt.at[src_v], k_v, sem)
        pltpu.async_copy(qt.at[dst_v], q_v, sem)
        pltpu.async_copy(vt.at[src_v], msg_v, sem)
        pltpu.async_copy(pe_hbm.at[pl.ds(base, C), pl.ds(col0, CW)],
                         pe_v, sem)

    def wait(b, c):
        k_v, q_v, pe_v, msg_v, s_v, src_v, dst_v, sem = bufs[b]
        base = c * C
        pltpu.make_async_copy(kt.at[src_v], k_v, sem).wait()
        pltpu.make_async_copy(qt.at[dst_v], q_v, sem).wait()
        pltpu.make_async_copy(vt.at[src_v], msg_v, sem).wait()
        pltpu.make_async_copy(pe_hbm.at[pl.ds(base, C), pl.ds(col0, CW)],
                              pe_v, sem).wait()

    def compute(b):
        k_v, q_v, pe_v, msg_v, s_v, src_v, dst_v, sem = bufs[b]

        @plsc.parallel_loop(0, C, unroll=4)
        def _(e):
            srow = jnp.zeros((16,), jnp.float32)
            for h in range(HH):
                sl = pl.ds(h * DH, DH)
                t = jnp.sum(k_v[e, sl] * q_v[e, sl] * pe_v[e, sl])
                sv = jnp.exp(jnp.clip(jnp.full((16,), t), -5.0, 5.0))
                msg_v[e, sl] = msg_v[e, sl] * sv
                srow = jnp.where(rows16 == h, sv, srow)
            s_v[e, :] = srow

        pltpu.sync_copy(msg_v, accw.at[dst_v], add=True)
        pltpu.sync_copy(s_v, accz.at[dst_v], add=True)

    for b in range(2):
        issue(b, b * NT + sid)

    def chunk_body(j, carry):
        for b in range(2):
            c = (2 * j + b) * NT + sid
            cn = c + 2 * NT

            @pl.when(c < NCHUNK)
            def _():
                wait(b, c)
                compute(b)

                @pl.when(cn < NCHUNK)
                def _():
                    issue(b, cn)

        return carry

    lax.fori_loop(0, JMAX, chunk_body, 0)
    plsc.subcore_barrier()

    pltpu.sync_copy(accw.at[pl.ds(r0, ROWS_PER_TILE)],
                    outw_hbm.at[cid, pl.ds(r0, ROWS_PER_TILE)])
    pltpu.sync_copy(accz.at[pl.ds(r0, ROWS_PER_TILE)],
                    outz_hbm.at[cid, pl.ds(r0, ROWS_PER_TILE)])

    @pl.when(sid == 0)
    def _():
        pltpu.sync_copy(accw.at[pl.ds(TAIL_ROW, TAIL_ROWS)],
                        outw_hbm.at[cid, pl.ds(TAIL_ROW, TAIL_ROWS)])
        pltpu.sync_copy(accz.at[pl.ds(TAIL_ROW, TAIL_ROWS)],
                        outz_hbm.at[cid, pl.ds(TAIL_ROW, TAIL_ROWS)])


_edge_call = pl.kernel(
    _edge_body,
    out_type=(jax.ShapeDtypeStruct((2, N, CW), jnp.float32),
              jax.ShapeDtypeStruct((2, N, 16), jnp.float32)),
    mesh=plsc.VectorSubcoreMesh(core_axis_name="c", subcore_axis_name="s"),
    compiler_params=pltpu.CompilerParams(use_tc_tiling_on_sc=False,
                                         needs_layout_passes=False),
    scratch_types=[
        pltpu.VMEM((C, CW), jnp.float32),
        pltpu.VMEM((C, CW), jnp.float32),
        pltpu.VMEM((C, CW), jnp.float32),
        pltpu.VMEM((C, CW), jnp.float32),
        pltpu.VMEM((C, CW), jnp.float32),
        pltpu.VMEM((C, CW), jnp.float32),
        pltpu.VMEM((C, CW), jnp.float32),
        pltpu.VMEM((C, CW), jnp.float32),
        pltpu.VMEM((C, 16), jnp.float32),
        pltpu.VMEM((C, 16), jnp.float32),
        pltpu.VMEM((C,), jnp.int32),
        pltpu.VMEM((C,), jnp.int32),
        pltpu.VMEM((C,), jnp.int32),
        pltpu.VMEM((C,), jnp.int32),
        pltpu.VMEM_SHARED((N, CW), jnp.float32),
        pltpu.VMEM_SHARED((N, 16), jnp.float32),
        pltpu.SemaphoreType.DMA,
        pltpu.SemaphoreType.DMA,
    ],
)


def kernel(x, edge_index, edge_attr, WQ, WK, WV, We):
    src = edge_index[0]
    dst = edge_index[1]
    wq = WQ * (1.0 / np.sqrt(np.float32(DH)))

    bn = 2000
    k, q, v = pl.pallas_call(
        _mm_nodes_body,
        grid=(N // bn,),
        in_specs=[
            pl.BlockSpec((bn, D), lambda i: (i, 0)),
            pl.BlockSpec((D, 128), lambda i: (0, 0)),
            pl.BlockSpec((D, 128), lambda i: (0, 0)),
            pl.BlockSpec((D, 128), lambda i: (0, 0)),
        ],
        out_specs=[
            pl.BlockSpec((2, bn, CW), lambda i: (0, i, 0)),
            pl.BlockSpec((2, bn, CW), lambda i: (0, i, 0)),
            pl.BlockSpec((2, bn, CW), lambda i: (0, i, 0)),
        ],
        out_shape=[
            jax.ShapeDtypeStruct((2, N, CW), jnp.float32),
            jax.ShapeDtypeStruct((2, N, CW), jnp.float32),
            jax.ShapeDtypeStruct((2, N, CW), jnp.float32),
        ],
    )(x, WK, wq, WV)

    be = 3200
    pe = pl.pallas_call(
        _mm_edges_body,
        grid=(E // be,),
        in_specs=[
            pl.BlockSpec((be, D), lambda i: (i, 0)),
            pl.BlockSpec((D, 128), lambda i: (0, 0)),
        ],
        out_specs=pl.BlockSpec((be, D), lambda i: (i, 0)),
        out_shape=jax.ShapeDtypeStruct((E, D), jnp.float32),
    )(edge_attr, We)

    zw = jnp.zeros((C, CW), jnp.float32)
    zz = jnp.zeros((C, 16), jnp.float32)
    outw, outz = _edge_call(k, q, v, pe, src, dst, zw, zz)

    bf = 2000
    out = pl.pallas_call(
        _final_body,
        grid=(N // bf,),
        in_specs=[
            pl.BlockSpec((2, bf, CW), lambda i: (0, i, 0)),
            pl.BlockSpec((2, bf, 16), lambda i: (0, i, 0)),
            pl.BlockSpec((32, 128), lambda i: (0, 0)),
        ],
        out_specs=pl.BlockSpec((bf, 128), lambda i: (i, 0)),
        out_shape=jax.ShapeDtypeStruct((N, 128), jnp.float32),
    )(outw, outz, jnp.asarray(_ZB))

    return out


# speedup vs baseline: 62.9826x; 1.0513x over previous
import jax
import jax.numpy as jnp
import numpy as np
from jax import lax
from jax.experimental import pallas as pl
from jax.experimental.pallas import tpu as pltpu
from jax.experimental.pallas import tpu_sc as plsc

N = 10000
E = 320000
D = 128
H = 8
DH = 16
HH = H // 2
CW = HH * DH

C = 80
NT = 16
NCHUNK = E // C
NSLOT = -(-NCHUNK // NT)
JMAX = -(-NSLOT // 6)
ROWS_PER_TILE = 624
TAIL_ROW = NT * ROWS_PER_TILE
TAIL_ROWS = N - TAIL_ROW

_ZB = np.zeros((32, 128), np.float32)
for _c in range(2):
    for _h in range(HH):
        _g = _c * HH + _h
        _ZB[_c * 16 + _h, _g * DH:(_g + 1) * DH] = 1.0
_ZB.setflags(write=False)


def _mm_proj_body(x_ref, ea_ref, wk_ref, wq_ref, wv_ref, we_ref,
                  kv_ref, q_ref, pe_ref):
    x = x_ref[...]
    rk = jnp.dot(x, wk_ref[...], preferred_element_type=jnp.float32)
    rq = jnp.dot(x, wq_ref[...], preferred_element_type=jnp.float32)
    rv = jnp.dot(x, wv_ref[...], preferred_element_type=jnp.float32)
    kv_ref[0] = jnp.concatenate([rk[:, :CW], rv[:, :CW]], axis=1)
    kv_ref[1] = jnp.concatenate([rk[:, CW:], rv[:, CW:]], axis=1)
    q_ref[0] = rq[:, :CW]
    q_ref[1] = rq[:, CW:]
    pe_ref[...] = jnp.dot(ea_ref[...], we_ref[...],
                          preferred_element_type=jnp.float32)


def _final_body(w_ref, z_ref, zb_ref, out_ref):
    w = jnp.concatenate([w_ref[0], w_ref[1]], axis=1)
    z = jnp.concatenate([z_ref[0], z_ref[1]], axis=1)
    zbig = jnp.dot(z, zb_ref[...], preferred_element_type=jnp.float32)
    out_ref[...] = w / (zbig + 1e-6)


def _edge_body(kv_hbm, q_hbm, pe_hbm, src_hbm, dst_hbm, zw_hbm, zz_hbm,
               outw_hbm, outz_hbm,
               kv_v0, kv_v1, q_v0, q_v1, pe_v0, pe_v1, src_v0, src_v1,
               msg_v0, msg_v1, msg_v2, s_v0, s_v1, s_v2,
               dst_v0, dst_v1, dst_v2,
               accw, accz, gsem0, gsem1, ssem0, ssem1, ssem2):
    cid = lax.axis_index("c")
    sid = lax.axis_index("s")
    kvt = kv_hbm.at[cid]
    qt = q_hbm.at[cid]
    col0 = pl.multiple_of(cid * CW, CW)
    gbufs = ((kv_v0, q_v0, pe_v0, src_v0, gsem0),
             (kv_v1, q_v1, pe_v1, src_v1, gsem1))
    mbufs = ((msg_v0, s_v0, dst_v0, ssem0),
             (msg_v1, s_v1, dst_v1, ssem1),
             (msg_v2, s_v2, dst_v2, ssem2))

    pltpu.sync_copy(zw_hbm, msg_v0)
    pltpu.sync_copy(zz_hbm, s_v0)
    r0 = sid * ROWS_PER_TILE
    for j in range(7):
        pltpu.sync_copy(msg_v0, accw.at[pl.ds(r0 + j * C, C)])
        pltpu.sync_copy(s_v0, accz.at[pl.ds(r0 + j * C, C)])
    pltpu.sync_copy(msg_v0.at[pl.ds(0, 64)], accw.at[pl.ds(r0 + 560, 64)])
    pltpu.sync_copy(s_v0.at[pl.ds(0, 64)], accz.at[pl.ds(r0 + 560, 64)])

    @pl.when(sid == 0)
    def _():
        pltpu.sync_copy(msg_v0.at[pl.ds(0, TAIL_ROWS)],
                        accw.at[pl.ds(TAIL_ROW, TAIL_ROWS)])
        pltpu.sync_copy(s_v0.at[pl.ds(0, TAIL_ROWS)],
                        accz.at[pl.ds(TAIL_ROW, TAIL_ROWS)])
    plsc.subcore_barrier()

    rows16 = lax.iota(jnp.int32, 16)

    def scatter_descs(m):
        msg_v, s_v, dst_v, ssem = mbufs[m]
        return (pltpu.make_async_copy(msg_v, accw.at[dst_v], ssem),
                pltpu.make_async_copy(s_v, accz.at[dst_v], ssem))

    def issue(b, m, c):
        kv_v, q_v, pe_v, src_v, gsem = gbufs[b]
        msg_v, s_v, dst_v, ssem = mbufs[m]
        base = c * C
        pltpu.sync_copy(src_hbm.at[pl.ds(base, C)], src_v)
        pltpu.sync_copy(dst_hbm.at[pl.ds(base, C)], dst_v)
        pltpu.async_copy(kvt.at[src_v], kv_v, gsem)
        pltpu.async_copy(qt.at[dst_v], q_v, gsem)
        pltpu.async_copy(pe_hbm.at[pl.ds(base, C), pl.ds(col0, CW)],
                         pe_v, gsem)

    def wait_gathers(b, m, c):
        kv_v, q_v, pe_v, src_v, gsem = gbufs[b]
        msg_v, s_v, dst_v, ssem = mbufs[m]
        base = c * C
        pltpu.make_async_copy(kvt.at[src_v], kv_v, gsem).wait()
        pltpu.make_async_copy(qt.at[dst_v], q_v, gsem).wait()
        pltpu.make_async_copy(pe_hbm.at[pl.ds(base, C), pl.ds(col0, CW)],
                              pe_v, gsem).wait()

    def compute(b, m, c):
        kv_v, q_v, pe_v, src_v, gsem = gbufs[b]
        msg_v, s_v, dst_v, ssem = mbufs[m]

        @plsc.parallel_loop(0, C, unroll=4)
        def _(e):
            srow = jnp.zeros((16,), jnp.float32)
            for h in range(HH):
                sl = pl.ds(h * DH, DH)
                t = jnp.sum(kv_v[e, sl] * q_v[e, sl] * pe_v[e, sl])
                sv = jnp.exp(jnp.clip(jnp.full((16,), t), -5.0, 5.0))
                msg_v[e, sl] = kv_v[e, pl.ds(CW + h * DH, DH)] * sv
                srow = jnp.where(rows16 == h, sv, srow)
            s_v[e, :] = srow

        pltpu.async_copy(msg_v, accw.at[dst_v], ssem, add=True)
        pltpu.async_copy(s_v, accz.at[dst_v], ssem, add=True)

        @pl.when(c + 3 * NT >= NCHUNK)
        def _():
            for d in scatter_descs(m):
                d.wait()

    for t in range(2):
        issue(t & 1, t % 3, t * NT + sid)

    def chunk_body(j, carry):
        for u in range(6):
            t = 6 * j + u
            c = t * NT + sid

            @pl.when(c < NCHUNK)
            def _():
                wait_gathers(u % 2, u % 3, c)
                compute(u % 2, u % 3, c)
                c2 = c + 2 * NT

                @pl.when(c2 < NCHUNK)
                def _():
                    if u == 0:
                        @pl.when(j > 0)
                        def _():
                            for d in scatter_descs((u + 2) % 3):
                                d.wait()
                    else:
                        for d in scatter_descs((u + 2) % 3):
                            d.wait()
                    issue(u % 2, (u + 2) % 3, c2)

        return carry

    lax.fori_loop(0, JMAX, chunk_body, 0)
    plsc.subcore_barrier()

    pltpu.sync_copy(accw.at[pl.ds(r0, ROWS_PER_TILE)],
                    outw_hbm.at[cid, pl.ds(r0, ROWS_PER_TILE)])
    pltpu.sync_copy(accz.at[pl.ds(r0, ROWS_PER_TILE)],
                    outz_hbm.at[cid, pl.ds(r0, ROWS_PER_TILE)])

    @pl.when(sid == 0)
    def _():
        pltpu.sync_copy(accw.at[pl.ds(TAIL_ROW, TAIL_ROWS)],
                        outw_hbm.at[cid, pl.ds(TAIL_ROW, TAIL_ROWS)])
        pltpu.sync_copy(accz.at[pl.ds(TAIL_ROW, TAIL_ROWS)],
                        outz_hbm.at[cid, pl.ds(TAIL_ROW, TAIL_ROWS)])


_edge_call = pl.kernel(
    _edge_body,
    out_type=(jax.ShapeDtypeStruct((2, N, CW), jnp.float32),
              jax.ShapeDtypeStruct((2, N, 16), jnp.float32)),
    mesh=plsc.VectorSubcoreMesh(core_axis_name="c", subcore_axis_name="s"),
    compiler_params=pltpu.CompilerParams(use_tc_tiling_on_sc=False,
                                         needs_layout_passes=False),
    scratch_types=[
        pltpu.VMEM((C, 2 * CW), jnp.float32),
        pltpu.VMEM((C, 2 * CW), jnp.float32),
        pltpu.VMEM((C, CW), jnp.float32),
        pltpu.VMEM((C, CW), jnp.float32),
        pltpu.VMEM((C, CW), jnp.float32),
        pltpu.VMEM((C, CW), jnp.float32),
        pltpu.VMEM((C,), jnp.int32),
        pltpu.VMEM((C,), jnp.int32),
        pltpu.VMEM((C, CW), jnp.float32),
        pltpu.VMEM((C, CW), jnp.float32),
        pltpu.VMEM((C, CW), jnp.float32),
        pltpu.VMEM((C, 16), jnp.float32),
        pltpu.VMEM((C, 16), jnp.float32),
        pltpu.VMEM((C, 16), jnp.float32),
        pltpu.VMEM((C,), jnp.int32),
        pltpu.VMEM((C,), jnp.int32),
        pltpu.VMEM((C,), jnp.int32),
        pltpu.VMEM_SHARED((N, CW), jnp.float32),
        pltpu.VMEM_SHARED((N, 16), jnp.float32),
        pltpu.SemaphoreType.DMA,
        pltpu.SemaphoreType.DMA,
        pltpu.SemaphoreType.DMA,
        pltpu.SemaphoreType.DMA,
        pltpu.SemaphoreType.DMA,
    ],
)


def kernel(x, edge_index, edge_attr, WQ, WK, WV, We):
    src = edge_index[0]
    dst = edge_index[1]
    wq = WQ * (1.0 / np.sqrt(np.float32(DH)))

    gp = 50
    bn = N // gp
    be = E // gp
    kv, q, pe = pl.pallas_call(
        _mm_proj_body,
        grid=(gp,),
        in_specs=[
            pl.BlockSpec((bn, D), lambda i: (i, 0)),
            pl.BlockSpec((be, D), lambda i: (i, 0)),
            pl.BlockSpec((D, 128), lambda i: (0, 0)),
            pl.BlockSpec((D, 128), lambda i: (0, 0)),
            pl.BlockSpec((D, 128), lambda i: (0, 0)),
            pl.BlockSpec((D, 128), lambda i: (0, 0)),
        ],
        out_specs=[
            pl.BlockSpec((2, bn, 2 * CW), lambda i: (0, i, 0)),
            pl.BlockSpec((2, bn, CW), lambda i: (0, i, 0)),
            pl.BlockSpec((be, D), lambda i: (i, 0)),
        ],
        out_shape=[
            jax.ShapeDtypeStruct((2, N, 2 * CW), jnp.float32),
            jax.ShapeDtypeStruct((2, N, CW), jnp.float32),
            jax.ShapeDtypeStruct((E, D), jnp.float32),
        ],
    )(x, edge_attr, WK, wq, WV, We)

    zw = jnp.zeros((C, CW), jnp.float32)
    zz = jnp.zeros((C, 16), jnp.float32)
    outw, outz = _edge_call(kv, q, pe, src, dst, zw, zz)

    bf = 2000
    out = pl.pallas_call(
        _final_body,
        grid=(N // bf,),
        in_specs=[
            pl.BlockSpec((2, bf, CW), lambda i: (0, i, 0)),
            pl.BlockSpec((2, bf, 16), lambda i: (0, i, 0)),
            pl.BlockSpec((32, 128), lambda i: (0, 0)),
        ],
        out_specs=pl.BlockSpec((bf, 128), lambda i: (i, 0)),
        out_shape=jax.ShapeDtypeStruct((N, 128), jnp.float32),
    )(outw, outz, jnp.asarray(_ZB))

    return out


# speedup vs baseline: 71.4539x; 1.1345x over previous
import jax
import jax.numpy as jnp
import numpy as np
from jax import lax
from jax.experimental import pallas as pl
from jax.experimental.pallas import tpu as pltpu
from jax.experimental.pallas import tpu_sc as plsc

N = 10000
E = 320000
D = 128
H = 8
DH = 16
HH = H // 2
CW = HH * DH

C = 128
NT = 16
NCHUNK = E // C
NSLOT = -(-NCHUNK // NT)
JMAX = -(-NSLOT // 6)
ROWS_PER_TILE = 624
TAIL_ROW = NT * ROWS_PER_TILE
TAIL_ROWS = N - TAIL_ROW

_ZB = np.zeros((32, 128), np.float32)
for _c in range(2):
    for _h in range(HH):
        _g = _c * HH + _h
        _ZB[_c * 16 + _h, _g * DH:(_g + 1) * DH] = 1.0
_ZB.setflags(write=False)


def _to_u32(a):
    b = jax.lax.bitcast_convert_type(a.astype(jnp.bfloat16), jnp.uint16)
    return b.astype(jnp.uint32)


def _mm_proj_body(x_ref, ea_ref, wk_ref, wq_ref, wv_ref, we_ref,
                  kv_ref, q_ref, pe_ref):
    x = x_ref[...]
    rk = jnp.dot(x, wk_ref[...], preferred_element_type=jnp.float32)
    rq = jnp.dot(x, wq_ref[...], preferred_element_type=jnp.float32)
    rv = jnp.dot(x, wv_ref[...], preferred_element_type=jnp.float32)
    kvw = _to_u32(rk) | (_to_u32(rv) << 16)
    kv_ref[0] = kvw[:, :CW]
    kv_ref[1] = kvw[:, CW:]
    qu = _to_u32(rq)
    for c in range(2):
        blocks = []
        for h2 in range(2):
            a = qu[:, c * CW + h2 * 32:c * CW + h2 * 32 + 16]
            b = qu[:, c * CW + h2 * 32 + 16:c * CW + h2 * 32 + 32]
            blocks.append(a | (b << 16))
        q_ref[c] = jnp.concatenate(blocks, axis=1)
    pe_ref[...] = jnp.dot(ea_ref[...], we_ref[...],
                          preferred_element_type=jnp.float32)


def _final_body(w_ref, z_ref, zb_ref, out_ref):
    w = jnp.concatenate([w_ref[0], w_ref[1]], axis=1)
    z = jnp.concatenate([z_ref[0], z_ref[1]], axis=1)
    zbig = jnp.dot(z, zb_ref[...], preferred_element_type=jnp.float32)
    out_ref[...] = w / (zbig + 1e-6)


def _edge_body(kv_hbm, q_hbm, pe_hbm, src_hbm, dst_hbm, zw_hbm, zz_hbm,
               outw_hbm, outz_hbm,
               kv_v0, kv_v1, q_v0, q_v1, pe_v0, pe_v1, src_v0, src_v1,
               msg_v0, msg_v1, msg_v2, s_v0, s_v1, s_v2,
               dst_v0, dst_v1, dst_v2,
               accw, accz, gsem0, gsem1, ssem0, ssem1, ssem2):
    cid = lax.axis_index("c")
    sid = lax.axis_index("s")
    kvt = kv_hbm.at[cid]
    qt = q_hbm.at[cid]
    col0 = pl.multiple_of(cid * CW, CW)
    gbufs = ((kv_v0, q_v0, pe_v0, src_v0, gsem0),
             (kv_v1, q_v1, pe_v1, src_v1, gsem1))
    mbufs = ((msg_v0, s_v0, dst_v0, ssem0),
             (msg_v1, s_v1, dst_v1, ssem1),
             (msg_v2, s_v2, dst_v2, ssem2))

    pltpu.sync_copy(zw_hbm, msg_v0)
    pltpu.sync_copy(zz_hbm, s_v0)
    r0 = sid * ROWS_PER_TILE
    for j in range(4):
        pltpu.sync_copy(msg_v0, accw.at[pl.ds(r0 + j * C, C)])
        pltpu.sync_copy(s_v0, accz.at[pl.ds(r0 + j * C, C)])
    pltpu.sync_copy(msg_v0.at[pl.ds(0, 112)], accw.at[pl.ds(r0 + 512, 112)])
    pltpu.sync_copy(s_v0.at[pl.ds(0, 112)], accz.at[pl.ds(r0 + 512, 112)])

    @pl.when(sid == 0)
    def _():
        pltpu.sync_copy(msg_v0.at[pl.ds(0, TAIL_ROWS)],
                        accw.at[pl.ds(TAIL_ROW, TAIL_ROWS)])
        pltpu.sync_copy(s_v0.at[pl.ds(0, TAIL_ROWS)],
                        accz.at[pl.ds(TAIL_ROW, TAIL_ROWS)])
    plsc.subcore_barrier()

    rows16 = lax.iota(jnp.int32, 16)

    def scatter_descs(m):
        msg_v, s_v, dst_v, ssem = mbufs[m]
        return (pltpu.make_async_copy(msg_v, accw.at[dst_v], ssem),
                pltpu.make_async_copy(s_v, accz.at[dst_v], ssem))

    def issue(b, m, c):
        kv_v, q_v, pe_v, src_v, gsem = gbufs[b]
        msg_v, s_v, dst_v, ssem = mbufs[m]
        base = c * C
        pltpu.sync_copy(src_hbm.at[pl.ds(base, C)], src_v)
        pltpu.sync_copy(dst_hbm.at[pl.ds(base, C)], dst_v)
        pltpu.async_copy(kvt.at[src_v], kv_v, gsem)
        pltpu.async_copy(qt.at[dst_v], q_v, gsem)
        pltpu.async_copy(pe_hbm.at[pl.ds(base, C), pl.ds(col0, CW)],
                         pe_v, gsem)

    def wait_gathers(b, m, c):
        kv_v, q_v, pe_v, src_v, gsem = gbufs[b]
        msg_v, s_v, dst_v, ssem = mbufs[m]
        base = c * C
        pltpu.make_async_copy(kvt.at[src_v], kv_v, gsem).wait()
        pltpu.make_async_copy(qt.at[dst_v], q_v, gsem).wait()
        pltpu.make_async_copy(pe_hbm.at[pl.ds(base, C), pl.ds(col0, CW)],
                              pe_v, gsem).wait()

    def compute(b, m, c):
        kv_v, q_v, pe_v, src_v, gsem = gbufs[b]
        msg_v, s_v, dst_v, ssem = mbufs[m]

        @plsc.parallel_loop(0, C, unroll=4)
        def _(e):
            srow = jnp.zeros((16,), jnp.float32)
            for h2 in range(2):
                qw = plsc.bitcast(q_v[e, pl.ds(h2 * 16, 16)], jnp.bfloat16)
                qa, qb = plsc.unpack(qw, format=plsc.PackFormat.INTERLEAVED,
                                     preferred_element_type=jnp.float32)
                for hh, qv in ((0, qa), (1, qb)):
                    h = h2 * 2 + hh
                    sl = pl.ds(h * DH, DH)
                    kw = plsc.bitcast(kv_v[e, sl], jnp.bfloat16)
                    kk, vv = plsc.unpack(
                        kw, format=plsc.PackFormat.INTERLEAVED,
                        preferred_element_type=jnp.float32)
                    t = jnp.sum(kk * qv * pe_v[e, sl])
                    sv = jnp.exp(jnp.clip(jnp.full((16,), t), -5.0, 5.0))
                    msg_v[e, sl] = vv * sv
                    srow = jnp.where(rows16 == h, sv, srow)
            s_v[e, :] = srow

        pltpu.async_copy(msg_v, accw.at[dst_v], ssem, add=True)
        pltpu.async_copy(s_v, accz.at[dst_v], ssem, add=True)

        @pl.when(c + 3 * NT >= NCHUNK)
        def _():
            for d in scatter_descs(m):
                d.wait()

    for t in range(2):
        issue(t & 1, t % 3, t * NT + sid)

    def chunk_body(j, carry):
        for u in range(6):
            t = 6 * j + u
            c = t * NT + sid

            @pl.when(c < NCHUNK)
            def _():
                wait_gathers(u % 2, u % 3, c)
                compute(u % 2, u % 3, c)
                c2 = c + 2 * NT

                @pl.when(c2 < NCHUNK)
                def _():
                    if u == 0:
                        @pl.when(j > 0)
                        def _():
                            for d in scatter_descs((u + 2) % 3):
                                d.wait()
                    else:
                        for d in scatter_descs((u + 2) % 3):
                            d.wait()
                    issue(u % 2, (u + 2) % 3, c2)

        return carry

    lax.fori_loop(0, JMAX, chunk_body, 0)
    plsc.subcore_barrier()

    pltpu.sync_copy(accw.at[pl.ds(r0, ROWS_PER_TILE)],
                    outw_hbm.at[cid, pl.ds(r0, ROWS_PER_TILE)])
    pltpu.sync_copy(accz.at[pl.ds(r0, ROWS_PER_TILE)],
                    outz_hbm.at[cid, pl.ds(r0, ROWS_PER_TILE)])

    @pl.when(sid == 0)
    def _():
        pltpu.sync_copy(accw.at[pl.ds(TAIL_ROW, TAIL_ROWS)],
                        outw_hbm.at[cid, pl.ds(TAIL_ROW, TAIL_ROWS)])
        pltpu.sync_copy(accz.at[pl.ds(TAIL_ROW, TAIL_ROWS)],
                        outz_hbm.at[cid, pl.ds(TAIL_ROW, TAIL_ROWS)])


_edge_call = pl.kernel(
    _edge_body,
    out_type=(jax.ShapeDtypeStruct((2, N, CW), jnp.float32),
              jax.ShapeDtypeStruct((2, N, 16), jnp.float32)),
    mesh=plsc.VectorSubcoreMesh(core_axis_name="c", subcore_axis_name="s"),
    compiler_params=pltpu.CompilerParams(use_tc_tiling_on_sc=False,
                                         needs_layout_passes=False),
    scratch_types=[
        pltpu.VMEM((C, CW), jnp.uint32),
        pltpu.VMEM((C, CW), jnp.uint32),
        pltpu.VMEM((C, 32), jnp.uint32),
        pltpu.VMEM((C, 32), jnp.uint32),
        pltpu.VMEM((C, CW), jnp.float32),
        pltpu.VMEM((C, CW), jnp.float32),
        pltpu.VMEM((C,), jnp.int32),
        pltpu.VMEM((C,), jnp.int32),
        pltpu.VMEM((C, CW), jnp.float32),
        pltpu.VMEM((C, CW), jnp.float32),
        pltpu.VMEM((C, CW), jnp.float32),
        pltpu.VMEM((C, 16), jnp.float32),
        pltpu.VMEM((C, 16), jnp.float32),
        pltpu.VMEM((C, 16), jnp.float32),
        pltpu.VMEM((C,), jnp.int32),
        pltpu.VMEM((C,), jnp.int32),
        pltpu.VMEM((C,), jnp.int32),
        pltpu.VMEM_SHARED((N, CW), jnp.float32),
        pltpu.VMEM_SHARED((N, 16), jnp.float32),
        pltpu.SemaphoreType.DMA,
        pltpu.SemaphoreType.DMA,
        pltpu.SemaphoreType.DMA,
        pltpu.SemaphoreType.DMA,
        pltpu.SemaphoreType.DMA,
    ],
)


def kernel(x, edge_index, edge_attr, WQ, WK, WV, We):
    src = edge_index[0]
    dst = edge_index[1]
    wq = WQ * (1.0 / np.sqrt(np.float32(DH)))

    gp = 50
    bn = N // gp
    be = E // gp
    kv, q, pe = pl.pallas_call(
        _mm_proj_body,
        grid=(gp,),
        in_specs=[
            pl.BlockSpec((bn, D), lambda i: (i, 0)),
            pl.BlockSpec((be, D), lambda i: (i, 0)),
            pl.BlockSpec((D, 128), lambda i: (0, 0)),
            pl.BlockSpec((D, 128), lambda i: (0, 0)),
            pl.BlockSpec((D, 128), lambda i: (0, 0)),
            pl.BlockSpec((D, 128), lambda i: (0, 0)),
        ],
        out_specs=[
            pl.BlockSpec((2, bn, CW), lambda i: (0, i, 0)),
            pl.BlockSpec((2, bn, 32), lambda i: (0, i, 0)),
            pl.BlockSpec((be, D), lambda i: (i, 0)),
        ],
        out_shape=[
            jax.ShapeDtypeStruct((2, N, CW), jnp.uint32),
            jax.ShapeDtypeStruct((2, N, 32), jnp.uint32),
            jax.ShapeDtypeStruct((E, D), jnp.float32),
        ],
    )(x, edge_attr, WK, wq, WV, We)

    zw = jnp.zeros((C, CW), jnp.float32)
    zz = jnp.zeros((C, 16), jnp.float32)
    outw, outz = _edge_call(kv, q, pe, src, dst, zw, zz)

    bf = 2000
    out = pl.pallas_call(
        _final_body,
        grid=(N // bf,),
        in_specs=[
            pl.BlockSpec((2, bf, CW), lambda i: (0, i, 0)),
            pl.BlockSpec((2, bf, 16), lambda i: (0, i, 0)),
            pl.BlockSpec((32, 128), lambda i: (0, 0)),
        ],
        out_specs=pl.BlockSpec((bf, 128), lambda i: (i, 0)),
        out_shape=jax.ShapeDtypeStruct((N, 128), jnp.float32),
    )(outw, outz, jnp.asarray(_ZB))

    return out
